# baseline (speedup 1.0000x reference)
import jax
import jax.numpy as jnp
from jax.experimental import pallas as pl
from jax.experimental.pallas import tpu as pltpu


def _leaky(x):
    return jnp.where(x >= 0, x, 0.01 * x)


def _edge_blocks(n):
    for b in (2000, 1000, 500, 250, 200, 125, 100, 64, 50, 32, 25, 16, 10, 8, 5, 4, 2):
        if n % b == 0 and b <= n:
            return b
    return n


def _row_block(n):
    for b in (1000, 500, 200, 100, 40, 8):
        if n % b == 0 and b <= n:
            return b
    return n


def _idx_spec(nb, b, ndim_grid=1):
    if ndim_grid == 1:
        imap = lambda s: (s, 0, 0)
    else:
        imap = lambda f, s: (s, 0, 0)
    return pl.BlockSpec((1, 1, b), imap, memory_space=pltpu.SMEM)


def _full(mem=None):
    if mem is None:
        return pl.BlockSpec(memory_space=pltpu.ANY)
    return pl.BlockSpec(memory_space=mem)


def _deg_relmax_body(i_ref, rel_ref, deg_ref, rmax_ref):
    step = pl.program_id(0)
    eb = i_ref.shape[2]

    @pl.when(step == 0)
    def _():
        deg_ref[...] = jnp.zeros_like(deg_ref)
        rmax_ref[0, 0] = jnp.int32(-1)

    def body(b, carry):
        i = i_ref[0, 0, b]
        r = rel_ref[0, 0, b]
        deg_ref[pl.ds(i, 1), :] += 1.0
        rmax_ref[0, 0] = jnp.maximum(rmax_ref[0, 0], r)
        return carry

    jax.lax.fori_loop(0, eb, body, 0, unroll=8)


def _deg_relmax(dst3, rel3, n):
    nb, _, eb = dst3.shape
    return pl.pallas_call(
        _deg_relmax_body,
        grid=(nb,),
        in_specs=[_idx_spec(nb, eb), _idx_spec(nb, eb)],
        out_specs=[
            pl.BlockSpec(memory_space=pltpu.VMEM),
            pl.BlockSpec(memory_space=pltpu.SMEM),
        ],
        out_shape=[
            jax.ShapeDtypeStruct((n, 1), jnp.float32),
            jax.ShapeDtypeStruct((1, 1), jnp.int32),
        ],
    )(dst3, rel3)


def _spmm_gcn_body(i_ref, j_ref, deg_ref, x_ref, out_ref, dis_ref):
    step = pl.program_id(0)
    eb = i_ref.shape[2]

    @pl.when(step == 0)
    def _():
        dis_ref[...] = jax.lax.rsqrt(deg_ref[...])
        out_ref[...] = jnp.zeros_like(out_ref)

    def body(b, carry):
        i = i_ref[0, 0, b]
        j = j_ref[0, 0, b]
        nrm = dis_ref[pl.ds(i, 1), :] * dis_ref[pl.ds(j, 1), :]
        out_ref[pl.ds(i, 1), :] += nrm * x_ref[pl.ds(j, 1), :]
        return carry

    jax.lax.fori_loop(0, eb, body, 0, unroll=4)


def _spmm_gcn(dst3, src3, deg, x):
    nb, _, eb = dst3.shape
    n, d = x.shape
    return pl.pallas_call(
        _spmm_gcn_body,
        grid=(nb,),
        in_specs=[
            _idx_spec(nb, eb), _idx_spec(nb, eb),
            pl.BlockSpec(memory_space=pltpu.VMEM),
            pl.BlockSpec(memory_space=pltpu.VMEM),
        ],
        out_specs=pl.BlockSpec(memory_space=pltpu.VMEM),
        out_shape=jax.ShapeDtypeStruct((n, d), jnp.float32),
        scratch_shapes=[pltpu.VMEM((n, 1), jnp.float32)],
    )(dst3, src3, deg, x)


def _gcn_hw_body(x_ref, agg_ref, gcnw_ref, hww_ref, hwb_ref, out_ref):
    dn = (((1,), (1,)), ((), ()))
    x = x_ref[...]
    x2 = jax.lax.dot_general(jnp.maximum(agg_ref[...], 0.0), gcnw_ref[...],
                             dn, preferred_element_type=jnp.float32)
    gate = jax.nn.sigmoid(
        jax.lax.dot_general(x, hww_ref[...], dn,
                            preferred_element_type=jnp.float32) + hwb_ref[...])
    out_ref[...] = gate * x2 + (1.0 - gate) * x


def _gcn_hw(x, agg, gcn_w, hw_w, hw_b):
    n, d = x.shape
    rb = _row_block(n)
    return pl.pallas_call(
        _gcn_hw_body,
        grid=(n // rb,),
        in_specs=[
            pl.BlockSpec((rb, d), lambda s: (s, 0)),
            pl.BlockSpec((rb, d), lambda s: (s, 0)),
            pl.BlockSpec((d, d), lambda s: (0, 0)),
            pl.BlockSpec((d, d), lambda s: (0, 0)),
            pl.BlockSpec((1, d), lambda s: (0, 0)),
        ],
        out_specs=pl.BlockSpec((rb, d), lambda s: (s, 0)),
        out_shape=jax.ShapeDtypeStruct((n, d), jnp.float32),
    )(x, agg, gcn_w, hw_w, hw_b.reshape(1, d))


def _proj_body(x_ref, w_ref, out_ref):
    out_ref[...] = jnp.dot(x_ref[...], w_ref[...],
                           preferred_element_type=jnp.float32)


def _proj(x, w):
    n, d = x.shape
    k = w.shape[1]
    rb = _row_block(n)
    return pl.pallas_call(
        _proj_body,
        grid=(n // rb,),
        in_specs=[
            pl.BlockSpec((rb, d), lambda s: (s, 0)),
            pl.BlockSpec((d, k), lambda s: (0, 0)),
        ],
        out_specs=pl.BlockSpec((rb, k), lambda s: (s, 0)),
        out_shape=jax.ShapeDtypeStruct((n, k), jnp.float32),
    )(x, w)


def _gatr_sum_body(j_ref, i_ref, q_ref, s_ref):
    step = pl.program_id(0)
    eb = j_ref.shape[2]

    @pl.when(step == 0)
    def _():
        s_ref[...] = jnp.zeros_like(s_ref)

    def body(b, carry):
        j = j_ref[0, 0, b]
        i = i_ref[0, 0, b]
        e = q_ref[pl.ds(i, 1), 0:1] + q_ref[pl.ds(j, 1), 1:2]
        s_ref[pl.ds(j, 1), :] += jnp.exp(_leaky(e))
        return carry

    jax.lax.fori_loop(0, eb, body, 0, unroll=8)


def _gatr_spmm_body(j_ref, i_ref, q_ref, s_ref, re_ref, out_ref):
    step = pl.program_id(0)
    nsteps = pl.num_programs(0)
    eb = j_ref.shape[2]

    @pl.when(step == 0)
    def _():
        out_ref[...] = jnp.zeros_like(out_ref)

    def body(b, carry):
        j = j_ref[0, 0, b]
        i = i_ref[0, 0, b]
        e = q_ref[pl.ds(i, 1), 0:1] + q_ref[pl.ds(j, 1), 1:2]
        a = jnp.exp(_leaky(e)) / (s_ref[pl.ds(j, 1), :] + 1e-16)
        out_ref[pl.ds(i, 1), :] += a * re_ref[pl.ds(j, 1), :]
        return carry

    jax.lax.fori_loop(0, eb, body, 0, unroll=4)

    @pl.when(step == nsteps - 1)
    def _():
        out_ref[...] = jnp.maximum(out_ref[...], 0.0)


def _gat_r(src3, dst3, q, re):
    nb, _, eb = src3.shape
    n, d = re.shape
    s = pl.pallas_call(
        _gatr_sum_body,
        grid=(nb,),
        in_specs=[_idx_spec(nb, eb), _idx_spec(nb, eb),
                  pl.BlockSpec(memory_space=pltpu.VMEM)],
        out_specs=pl.BlockSpec(memory_space=pltpu.VMEM),
        out_shape=jax.ShapeDtypeStruct((n, 1), jnp.float32),
    )(src3, dst3, q)
    return pl.pallas_call(
        _gatr_spmm_body,
        grid=(nb,),
        in_specs=[_idx_spec(nb, eb), _idx_spec(nb, eb),
                  pl.BlockSpec(memory_space=pltpu.VMEM),
                  pl.BlockSpec(memory_space=pltpu.VMEM),
                  pl.BlockSpec(memory_space=pltpu.VMEM)],
        out_specs=pl.BlockSpec(memory_space=pltpu.VMEM),
        out_shape=jax.ShapeDtypeStruct((n, d), jnp.float32),
    )(src3, dst3, q, s, re)


def _rel_proj_body(re_ref, wb_ref, ar_ref, er_ref, pr_ref, xr_ref):
    rel = re_ref[...]
    er = _leaky(rel)
    er_ref[...] = er
    pr_ref[...] = jnp.dot(er, wb_ref[...], preferred_element_type=jnp.float32)
    xr_ref[...] = jnp.dot(rel, ar_ref[...], preferred_element_type=jnp.float32)


def _rel_proj(rel_emb, wb, ar):
    n, d = rel_emb.shape
    return pl.pallas_call(
        _rel_proj_body,
        out_shape=[
            jax.ShapeDtypeStruct((n, d), jnp.float32),
            jax.ShapeDtypeStruct((n, 1), jnp.float32),
            jax.ShapeDtypeStruct((n, 1), jnp.float32),
        ],
    )(rel_emb, wb.reshape(d, 1), ar.reshape(d, 1))


def _ga_proj_body(x_ref, w_ref, ef_ref, p_ref):
    ef = _leaky(x_ref[...])
    ef_ref[...] = ef
    p_ref[...] = jnp.dot(ef, w_ref[...], preferred_element_type=jnp.float32)


def _ga_proj(x, wac):
    n, d = x.shape
    rb = _row_block(n)
    return pl.pallas_call(
        _ga_proj_body,
        grid=(n // rb,),
        in_specs=[
            pl.BlockSpec((rb, d), lambda s: (s, 0)),
            pl.BlockSpec((d, 2), lambda s: (0, 0)),
        ],
        out_specs=[
            pl.BlockSpec((rb, d), lambda s: (s, 0)),
            pl.BlockSpec((rb, 2), lambda s: (s, 0)),
        ],
        out_shape=[
            jax.ShapeDtypeStruct((n, d), jnp.float32),
            jax.ShapeDtypeStruct((n, 2), jnp.float32),
        ],
    )(x, wac)


def _ga_sum_body(i_ref, j_ref, r_ref, p_ref, pr_ref, s_ref):
    step = pl.program_id(0)
    eb = i_ref.shape[2]

    @pl.when(step == 0)
    def _():
        s_ref[...] = jnp.zeros_like(s_ref)

    def body(b, carry):
        i = i_ref[0, 0, b]
        j = j_ref[0, 0, b]
        r = r_ref[0, 0, b]
        e = (p_ref[pl.ds(i, 1), 0:1] + pr_ref[pl.ds(r, 1), :]
             + p_ref[pl.ds(j, 1), 1:2])
        s_ref[pl.ds(i, 1), :] += jnp.exp(e)
        return carry

    jax.lax.fori_loop(0, eb, body, 0, unroll=8)


def _ga_scatter_ab_body(i_ref, j_ref, r_ref, p_ref, pr_ref, s_ref,
                        er_ref, t1_ref, outb_ref):
    step = pl.program_id(0)
    eb = i_ref.shape[2]

    @pl.when(step == 0)
    def _():
        t1_ref[...] = jnp.zeros_like(t1_ref)
        outb_ref[...] = jnp.zeros_like(outb_ref)

    def body(b, carry):
        i = i_ref[0, 0, b]
        j = j_ref[0, 0, b]
        r = r_ref[0, 0, b]
        e = (p_ref[pl.ds(i, 1), 0:1] + pr_ref[pl.ds(r, 1), :]
             + p_ref[pl.ds(j, 1), 1:2])
        a = jnp.exp(e) / (s_ref[pl.ds(i, 1), :] + 1e-16)
        t1_ref[pl.ds(i, 1), :] += a
        outb_ref[pl.ds(i, 1), :] += a * er_ref[pl.ds(r, 1), :]
        return carry

    jax.lax.fori_loop(0, eb, body, 0, unroll=4)


def _ga_scatter_c_body(i_ref, j_ref, r_ref, p_ref, pr_ref, s_ref, ef_ref,
                       outc_ref):
    step = pl.program_id(0)
    eb = i_ref.shape[2]

    @pl.when(step == 0)
    def _():
        outc_ref[...] = jnp.zeros_like(outc_ref)

    def body(b, carry):
        i = i_ref[0, 0, b]
        j = j_ref[0, 0, b]
        r = r_ref[0, 0, b]
        e = (p_ref[pl.ds(i, 1), 0:1] + pr_ref[pl.ds(r, 1), :]
             + p_ref[pl.ds(j, 1), 1:2])
        a = jnp.exp(e) / (s_ref[pl.ds(i, 1), :] + 1e-16)
        outc_ref[pl.ds(i, 1), :] += a * ef_ref[pl.ds(j, 1), :]
        return carry

    jax.lax.fori_loop(0, eb, body, 0, unroll=4)


def _scale_rows_body(x_ref, t_ref, out_ref):
    out_ref[...] = x_ref[...] * t_ref[...]


def _scale_rows(x, t):
    n, d = x.shape
    rb = _row_block(n)
    return pl.pallas_call(
        _scale_rows_body,
        grid=(n // rb,),
        in_specs=[pl.BlockSpec((rb, d), lambda s: (s, 0)),
                  pl.BlockSpec((rb, 1), lambda s: (s, 0))],
        out_specs=pl.BlockSpec((rb, d), lambda s: (s, 0)),
        out_shape=jax.ShapeDtypeStruct((n, d), jnp.float32),
    )(x, t)


def _graph_att(i3, j3, r3, p, pr, ef, er):
    nb, _, eb = i3.shape
    n, d = ef.shape
    dr = er.shape[1]
    vm = pl.BlockSpec(memory_space=pltpu.VMEM)
    s = pl.pallas_call(
        _ga_sum_body,
        grid=(nb,),
        in_specs=[_idx_spec(nb, eb), _idx_spec(nb, eb), _idx_spec(nb, eb),
                  vm, vm],
        out_specs=vm,
        out_shape=jax.ShapeDtypeStruct((n, 1), jnp.float32),
    )(i3, j3, r3, p, pr)
    t1, outb = pl.pallas_call(
        _ga_scatter_ab_body,
        grid=(nb,),
        in_specs=[_idx_spec(nb, eb), _idx_spec(nb, eb), _idx_spec(nb, eb),
                  vm, vm, vm, vm],
        out_specs=[vm, vm],
        out_shape=[
            jax.ShapeDtypeStruct((n, 1), jnp.float32),
            jax.ShapeDtypeStruct((n, dr), jnp.float32),
        ],
    )(i3, j3, r3, p, pr, s, er)
    outc = pl.pallas_call(
        _ga_scatter_c_body,
        grid=(nb,),
        in_specs=[_idx_spec(nb, eb), _idx_spec(nb, eb), _idx_spec(nb, eb),
                  vm, vm, vm, vm],
        out_specs=vm,
        out_shape=jax.ShapeDtypeStruct((n, d), jnp.float32),
    )(i3, j3, r3, p, pr, s, ef)
    outa = _scale_rows(ef, t1)
    return outa, outb, outc


def _gat_sum_body(i_ref, j_ref, r_ref, q_ref, xr_ref, s_ref):
    step = pl.program_id(0)
    eb = i_ref.shape[2]

    @pl.when(step == 0)
    def _():
        s_ref[...] = jnp.zeros_like(s_ref)

    def body(b, carry):
        i = i_ref[0, 0, b]
        j = j_ref[0, 0, b]
        r = r_ref[0, 0, b]

        @pl.when(i != j)
        def _():
            e = (q_ref[pl.ds(i, 1), 0:1] + q_ref[pl.ds(j, 1), 1:2]
                 + xr_ref[pl.ds(r, 1), :])
            s_ref[pl.ds(i, 1), :] += jnp.exp(_leaky(e))

        return carry

    jax.lax.fori_loop(0, eb, body, 0, unroll=8)


def _gat_spmm_body(i_ref, j_ref, r_ref, q_ref, xr_ref, s_ref, x_ref, out_ref):
    estep = pl.program_id(0)
    nsteps = pl.num_programs(0)
    eb = i_ref.shape[2]

    @pl.when(estep == 0)
    def _():
        out_ref[...] = jnp.zeros_like(out_ref)

    def body(b, carry):
        i = i_ref[0, 0, b]
        j = j_ref[0, 0, b]
        r = r_ref[0, 0, b]

        @pl.when(i != j)
        def _():
            e = (q_ref[pl.ds(i, 1), 0:1] + q_ref[pl.ds(j, 1), 1:2]
                 + xr_ref[pl.ds(r, 1), :])
            a = jnp.exp(_leaky(e)) / (s_ref[pl.ds(i, 1), :] + 1e-16)
            out_ref[pl.ds(i, 1), :] += a * x_ref[pl.ds(j, 1), :]

        return carry

    jax.lax.fori_loop(0, eb, body, 0, unroll=4)

    @pl.when(estep == nsteps - 1)
    def _():
        out_ref[...] = jnp.maximum(out_ref[...], 0.0)


def _gat(i3, j3, r3, q, xr, x):
    nb, _, eb = i3.shape
    n, d = x.shape
    vm = pl.BlockSpec(memory_space=pltpu.VMEM)
    s = pl.pallas_call(
        _gat_sum_body,
        grid=(nb,),
        in_specs=[_idx_spec(nb, eb), _idx_spec(nb, eb), _idx_spec(nb, eb),
                  vm, vm],
        out_specs=vm,
        out_shape=jax.ShapeDtypeStruct((n, 1), jnp.float32),
    )(i3, j3, r3, q, xr)
    cb = 512 if d % 512 == 0 else d
    outs = []
    for c0 in range(0, d, cb):
        outs.append(pl.pallas_call(
            _gat_spmm_body,
            grid=(nb,),
            in_specs=[_idx_spec(nb, eb), _idx_spec(nb, eb), _idx_spec(nb, eb),
                      vm, vm, vm, vm],
            out_specs=vm,
            out_shape=jax.ShapeDtypeStruct((n, cb), jnp.float32),
        )(i3, j3, r3, q, xr, s, x[:, c0:c0 + cb]))
    return outs[0] if len(outs) == 1 else jnp.concatenate(outs, axis=1)


def kernel(x_e, edge_index, rel, edge_index_all, rel_all,
           line_graph_index_out, line_graph_val_out,
           line_graph_index_in, line_graph_val_in,
           rel_emb1, rel_emb2, gcn1_w, gcn2_w,
           hw1_w, hw1_b, hw2_w, hw2_b, ww1_w,
           gat_ai, gat_aj, gat_ar, gatr_ai, gatr_aj):
    n, d = x_e.shape
    e_all = edge_index_all.shape[1]
    e_lg = line_graph_index_out.shape[1]
    eb = _edge_blocks(e_all)
    eb_lg = _edge_blocks(e_lg)

    def blk(a, b):
        return a.astype(jnp.int32).reshape(-1, 1, b)

    src3 = blk(edge_index_all[0], eb)
    dst3 = blk(edge_index_all[1], eb)
    rall3 = blk(rel_all, eb)
    rel3 = blk(rel, eb)

    deg, rmax = _deg_relmax(dst3, rel3, n)
    agg1 = _spmm_gcn(dst3, src3, deg, x_e)
    x1 = _gcn_hw(x_e, agg1, gcn1_w, hw1_w, hw1_b)
    agg2 = _spmm_gcn(dst3, src3, deg, x1)
    x2 = _gcn_hw(x1, agg2, gcn2_w, hw2_w, hw2_b)

    re = jnp.where(rmax[0, 0] + 1 == rel_emb1.shape[0], rel_emb1, rel_emb2)
    qr = _proj(re, jnp.stack([gatr_ai, gatr_aj], axis=1))
    lo_src3 = blk(line_graph_index_out[0], eb_lg)
    lo_dst3 = blk(line_graph_index_out[1], eb_lg)
    li_src3 = blk(line_graph_index_in[0], eb_lg)
    li_dst3 = blk(line_graph_index_in[1], eb_lg)
    rel_out = _gat_r(lo_src3, lo_dst3, qr, re)
    rel_in = _gat_r(li_src3, li_dst3, qr, re)
    rel_emb = jnp.concatenate([rel_out, rel_in], axis=0)

    er, pr, xr = _rel_proj(rel_emb, ww1_w[d:d + rel_emb.shape[1]], gat_ar)
    wac = jnp.stack([ww1_w[:d], ww1_w[d + rel_emb.shape[1]:]], axis=1)
    ef, p = _ga_proj(x2, wac)
    outa, outb, outc = _graph_att(src3, dst3, rall3, p, pr, ef, er)
    x_wjq = jnp.concatenate([x2, outa, outb, outc], axis=1)

    q2 = _proj(x_wjq, jnp.stack([gat_ai, gat_aj], axis=1))
    d_wjq = x_wjq.shape[1]
    dpad = -(-d_wjq // 512) * 512 if d_wjq > 512 else d_wjq
    x_in = jnp.pad(x_wjq, ((0, 0), (0, dpad - d_wjq))) if dpad != d_wjq else x_wjq
    gout = _gat(dst3, src3, rall3, q2, xr, x_in)[:, :d_wjq]
    return jnp.concatenate([x_wjq, gout], axis=1)


# speedup vs baseline: 1.0207x; 1.0207x over previous
import jax
import jax.numpy as jnp
from jax.experimental import pallas as pl
from jax.experimental.pallas import tpu as pltpu


def _leaky(x):
    return jnp.where(x >= 0, x, 0.01 * x)


def _edge_blocks(n):
    for b in (2000, 1000, 500, 250, 200, 125, 100, 64, 50, 32, 25, 16, 10, 8, 5, 4, 2):
        if n % b == 0 and b <= n:
            return b
    return n


def _row_block(n):
    for b in (1000, 500, 200, 100, 40, 8):
        if n % b == 0 and b <= n:
            return b
    return n


def _idx_spec(nb, b, ndim_grid=1):
    if ndim_grid == 1:
        imap = lambda s: (s, 0, 0)
    else:
        imap = lambda f, s: (s, 0, 0)
    return pl.BlockSpec((1, 1, b), imap, memory_space=pltpu.SMEM)


def _full(mem=None):
    if mem is None:
        return pl.BlockSpec(memory_space=pltpu.ANY)
    return pl.BlockSpec(memory_space=mem)


def _deg_relmax_body(i_ref, rel_ref, deg_ref, rmax_ref):
    step = pl.program_id(0)
    eb = i_ref.shape[2]

    @pl.when(step == 0)
    def _():
        deg_ref[...] = jnp.zeros_like(deg_ref)
        rmax_ref[0, 0] = jnp.int32(-1)

    def body(b, carry):
        i = i_ref[0, 0, b]
        r = rel_ref[0, 0, b]
        deg_ref[pl.ds(i, 1), :] += 1.0
        rmax_ref[0, 0] = jnp.maximum(rmax_ref[0, 0], r)
        return carry

    jax.lax.fori_loop(0, eb, body, 0, unroll=8)


def _deg_relmax(dst3, rel3, n):
    nb, _, eb = dst3.shape
    return pl.pallas_call(
        _deg_relmax_body,
        grid=(nb,),
        in_specs=[_idx_spec(nb, eb), _idx_spec(nb, eb)],
        out_specs=[
            pl.BlockSpec(memory_space=pltpu.VMEM),
            pl.BlockSpec(memory_space=pltpu.SMEM),
        ],
        out_shape=[
            jax.ShapeDtypeStruct((n, 1), jnp.float32),
            jax.ShapeDtypeStruct((1, 1), jnp.int32),
        ],
    )(dst3, rel3)


def _spmm_gcn_body(i_ref, j_ref, deg_ref, x_ref, out_ref, dis_ref):
    step = pl.program_id(0)
    eb = i_ref.shape[2]

    @pl.when(step == 0)
    def _():
        dis_ref[...] = jax.lax.rsqrt(deg_ref[...])
        out_ref[...] = jnp.zeros_like(out_ref)

    def body(b, carry):
        i = i_ref[0, 0, b]
        j = j_ref[0, 0, b]
        nrm = dis_ref[pl.ds(i, 1), :] * dis_ref[pl.ds(j, 1), :]
        out_ref[pl.ds(i, 1), :] += nrm * x_ref[pl.ds(j, 1), :]
        return carry

    jax.lax.fori_loop(0, eb, body, 0, unroll=4)


def _spmm_gcn(dst3, src3, deg, x):
    nb, _, eb = dst3.shape
    n, d = x.shape
    return pl.pallas_call(
        _spmm_gcn_body,
        grid=(nb,),
        in_specs=[
            _idx_spec(nb, eb), _idx_spec(nb, eb),
            pl.BlockSpec(memory_space=pltpu.VMEM),
            pl.BlockSpec(memory_space=pltpu.VMEM),
        ],
        out_specs=pl.BlockSpec(memory_space=pltpu.VMEM),
        out_shape=jax.ShapeDtypeStruct((n, d), jnp.float32),
        scratch_shapes=[pltpu.VMEM((n, 1), jnp.float32)],
    )(dst3, src3, deg, x)


def _gcn_hw_body(x_ref, agg_ref, gcnw_ref, hww_ref, hwb_ref, out_ref):
    dn = (((1,), (1,)), ((), ()))
    x = x_ref[...]
    x2 = jax.lax.dot_general(jnp.maximum(agg_ref[...], 0.0), gcnw_ref[...],
                             dn, preferred_element_type=jnp.float32)
    gate = jax.nn.sigmoid(
        jax.lax.dot_general(x, hww_ref[...], dn,
                            preferred_element_type=jnp.float32) + hwb_ref[...])
    out_ref[...] = gate * x2 + (1.0 - gate) * x


def _gcn_hw(x, agg, gcn_w, hw_w, hw_b):
    n, d = x.shape
    rb = _row_block(n)
    return pl.pallas_call(
        _gcn_hw_body,
        grid=(n // rb,),
        in_specs=[
            pl.BlockSpec((rb, d), lambda s: (s, 0)),
            pl.BlockSpec((rb, d), lambda s: (s, 0)),
            pl.BlockSpec((d, d), lambda s: (0, 0)),
            pl.BlockSpec((d, d), lambda s: (0, 0)),
            pl.BlockSpec((1, d), lambda s: (0, 0)),
        ],
        out_specs=pl.BlockSpec((rb, d), lambda s: (s, 0)),
        out_shape=jax.ShapeDtypeStruct((n, d), jnp.float32),
    )(x, agg, gcn_w, hw_w, hw_b.reshape(1, d))


def _proj_body(x_ref, w_ref, out_ref):
    out_ref[...] = jnp.dot(x_ref[...], w_ref[...],
                           preferred_element_type=jnp.float32)


def _proj(x, w):
    n, d = x.shape
    k = w.shape[1]
    rb = _row_block(n)
    return pl.pallas_call(
        _proj_body,
        grid=(n // rb,),
        in_specs=[
            pl.BlockSpec((rb, d), lambda s: (s, 0)),
            pl.BlockSpec((d, k), lambda s: (0, 0)),
        ],
        out_specs=pl.BlockSpec((rb, k), lambda s: (s, 0)),
        out_shape=jax.ShapeDtypeStruct((n, k), jnp.float32),
    )(x, w)


def _gatr_sum_body(j_ref, i_ref, q_ref, s_ref):
    step = pl.program_id(0)
    nsteps = pl.num_programs(0)
    eb = j_ref.shape[2]

    @pl.when(step == 0)
    def _():
        s_ref[...] = jnp.zeros_like(s_ref)

    def body(b, carry):
        j = j_ref[0, 0, b]
        i = i_ref[0, 0, b]
        e = q_ref[pl.ds(i, 1), 0:1] + q_ref[pl.ds(j, 1), 1:2]
        s_ref[pl.ds(j, 1), :] += jnp.exp(_leaky(e))
        return carry

    jax.lax.fori_loop(0, eb, body, 0, unroll=8)

    @pl.when(step == nsteps - 1)
    def _():
        s_ref[...] = 1.0 / (s_ref[...] + 1e-16)


def _gatr_spmm_body(j_ref, i_ref, q_ref, s_ref, re_ref, out_ref):
    step = pl.program_id(0)
    nsteps = pl.num_programs(0)
    eb = j_ref.shape[2]

    @pl.when(step == 0)
    def _():
        out_ref[...] = jnp.zeros_like(out_ref)

    def body(b, carry):
        j = j_ref[0, 0, b]
        i = i_ref[0, 0, b]
        e = q_ref[pl.ds(i, 1), 0:1] + q_ref[pl.ds(j, 1), 1:2]
        a = jnp.exp(_leaky(e)) * s_ref[pl.ds(j, 1), :]
        out_ref[pl.ds(i, 1), :] += a * re_ref[pl.ds(j, 1), :]
        return carry

    jax.lax.fori_loop(0, eb, body, 0, unroll=8)

    @pl.when(step == nsteps - 1)
    def _():
        out_ref[...] = jnp.maximum(out_ref[...], 0.0)


def _gat_r(src3, dst3, q, re):
    nb, _, eb = src3.shape
    n, d = re.shape
    s = pl.pallas_call(
        _gatr_sum_body,
        grid=(nb,),
        in_specs=[_idx_spec(nb, eb), _idx_spec(nb, eb),
                  pl.BlockSpec(memory_space=pltpu.VMEM)],
        out_specs=pl.BlockSpec(memory_space=pltpu.VMEM),
        out_shape=jax.ShapeDtypeStruct((n, 1), jnp.float32),
    )(src3, dst3, q)
    return pl.pallas_call(
        _gatr_spmm_body,
        grid=(nb,),
        in_specs=[_idx_spec(nb, eb), _idx_spec(nb, eb),
                  pl.BlockSpec(memory_space=pltpu.VMEM),
                  pl.BlockSpec(memory_space=pltpu.VMEM),
                  pl.BlockSpec(memory_space=pltpu.VMEM)],
        out_specs=pl.BlockSpec(memory_space=pltpu.VMEM),
        out_shape=jax.ShapeDtypeStruct((n, d), jnp.float32),
    )(src3, dst3, q, s, re)


def _rel_proj_body(re_ref, wb_ref, ar_ref, er_ref, pr_ref, xr_ref):
    rel = re_ref[...]
    er = _leaky(rel)
    er_ref[...] = er
    pr_ref[...] = jnp.dot(er, wb_ref[...], preferred_element_type=jnp.float32)
    xr_ref[...] = jnp.dot(rel, ar_ref[...], preferred_element_type=jnp.float32)


def _rel_proj(rel_emb, wb, ar):
    n, d = rel_emb.shape
    return pl.pallas_call(
        _rel_proj_body,
        out_shape=[
            jax.ShapeDtypeStruct((n, d), jnp.float32),
            jax.ShapeDtypeStruct((n, 1), jnp.float32),
            jax.ShapeDtypeStruct((n, 1), jnp.float32),
        ],
    )(rel_emb, wb.reshape(d, 1), ar.reshape(d, 1))


def _ga_proj_body(x_ref, w_ref, ef_ref, p_ref):
    ef = _leaky(x_ref[...])
    ef_ref[...] = ef
    p_ref[...] = jnp.dot(ef, w_ref[...], preferred_element_type=jnp.float32)


def _ga_proj(x, wac):
    n, d = x.shape
    rb = _row_block(n)
    return pl.pallas_call(
        _ga_proj_body,
        grid=(n // rb,),
        in_specs=[
            pl.BlockSpec((rb, d), lambda s: (s, 0)),
            pl.BlockSpec((d, 2), lambda s: (0, 0)),
        ],
        out_specs=[
            pl.BlockSpec((rb, d), lambda s: (s, 0)),
            pl.BlockSpec((rb, 2), lambda s: (s, 0)),
        ],
        out_shape=[
            jax.ShapeDtypeStruct((n, d), jnp.float32),
            jax.ShapeDtypeStruct((n, 2), jnp.float32),
        ],
    )(x, wac)


def _ga_sum_body(i_ref, j_ref, r_ref, p_ref, pr_ref, s_ref):
    step = pl.program_id(0)
    eb = i_ref.shape[2]

    @pl.when(step == 0)
    def _():
        s_ref[...] = jnp.zeros_like(s_ref)

    def body(b, carry):
        i = i_ref[0, 0, b]
        j = j_ref[0, 0, b]
        r = r_ref[0, 0, b]
        e = (p_ref[pl.ds(i, 1), 0:1] + pr_ref[pl.ds(r, 1), :]
             + p_ref[pl.ds(j, 1), 1:2])
        s_ref[pl.ds(i, 1), :] += jnp.exp(e)
        return carry

    jax.lax.fori_loop(0, eb, body, 0, unroll=8)


def _ga_scatter_ab_body(i_ref, j_ref, r_ref, p_ref, pr_ref, s_ref,
                        er_ref, t1_ref, outb_ref):
    step = pl.program_id(0)
    eb = i_ref.shape[2]

    @pl.when(step == 0)
    def _():
        t1_ref[...] = jnp.zeros_like(t1_ref)
        outb_ref[...] = jnp.zeros_like(outb_ref)

    def body(b, carry):
        i = i_ref[0, 0, b]
        j = j_ref[0, 0, b]
        r = r_ref[0, 0, b]
        e = (p_ref[pl.ds(i, 1), 0:1] + pr_ref[pl.ds(r, 1), :]
             + p_ref[pl.ds(j, 1), 1:2])
        a = jnp.exp(e) / (s_ref[pl.ds(i, 1), :] + 1e-16)
        t1_ref[pl.ds(i, 1), :] += a
        outb_ref[pl.ds(i, 1), :] += a * er_ref[pl.ds(r, 1), :]
        return carry

    jax.lax.fori_loop(0, eb, body, 0, unroll=4)


def _ga_scatter_c_body(i_ref, j_ref, r_ref, p_ref, pr_ref, s_ref, ef_ref,
                       outc_ref):
    step = pl.program_id(0)
    eb = i_ref.shape[2]

    @pl.when(step == 0)
    def _():
        outc_ref[...] = jnp.zeros_like(outc_ref)

    def body(b, carry):
        i = i_ref[0, 0, b]
        j = j_ref[0, 0, b]
        r = r_ref[0, 0, b]
        e = (p_ref[pl.ds(i, 1), 0:1] + pr_ref[pl.ds(r, 1), :]
             + p_ref[pl.ds(j, 1), 1:2])
        a = jnp.exp(e) / (s_ref[pl.ds(i, 1), :] + 1e-16)
        outc_ref[pl.ds(i, 1), :] += a * ef_ref[pl.ds(j, 1), :]
        return carry

    jax.lax.fori_loop(0, eb, body, 0, unroll=4)


def _scale_rows_body(x_ref, t_ref, out_ref):
    out_ref[...] = x_ref[...] * t_ref[...]


def _scale_rows(x, t):
    n, d = x.shape
    rb = _row_block(n)
    return pl.pallas_call(
        _scale_rows_body,
        grid=(n // rb,),
        in_specs=[pl.BlockSpec((rb, d), lambda s: (s, 0)),
                  pl.BlockSpec((rb, 1), lambda s: (s, 0))],
        out_specs=pl.BlockSpec((rb, d), lambda s: (s, 0)),
        out_shape=jax.ShapeDtypeStruct((n, d), jnp.float32),
    )(x, t)


def _graph_att(i3, j3, r3, p, pr, ef, er):
    nb, _, eb = i3.shape
    n, d = ef.shape
    dr = er.shape[1]
    vm = pl.BlockSpec(memory_space=pltpu.VMEM)
    s = pl.pallas_call(
        _ga_sum_body,
        grid=(nb,),
        in_specs=[_idx_spec(nb, eb), _idx_spec(nb, eb), _idx_spec(nb, eb),
                  vm, vm],
        out_specs=vm,
        out_shape=jax.ShapeDtypeStruct((n, 1), jnp.float32),
    )(i3, j3, r3, p, pr)
    t1, outb = pl.pallas_call(
        _ga_scatter_ab_body,
        grid=(nb,),
        in_specs=[_idx_spec(nb, eb), _idx_spec(nb, eb), _idx_spec(nb, eb),
                  vm, vm, vm, vm],
        out_specs=[vm, vm],
        out_shape=[
            jax.ShapeDtypeStruct((n, 1), jnp.float32),
            jax.ShapeDtypeStruct((n, dr), jnp.float32),
        ],
    )(i3, j3, r3, p, pr, s, er)
    outc = pl.pallas_call(
        _ga_scatter_c_body,
        grid=(nb,),
        in_specs=[_idx_spec(nb, eb), _idx_spec(nb, eb), _idx_spec(nb, eb),
                  vm, vm, vm, vm],
        out_specs=vm,
        out_shape=jax.ShapeDtypeStruct((n, d), jnp.float32),
    )(i3, j3, r3, p, pr, s, ef)
    outa = _scale_rows(ef, t1)
    return outa, outb, outc


def _gat_sum_body(i_ref, j_ref, r_ref, q_ref, xr_ref, s_ref):
    step = pl.program_id(0)
    eb = i_ref.shape[2]

    @pl.when(step == 0)
    def _():
        s_ref[...] = jnp.zeros_like(s_ref)

    def body(b, carry):
        i = i_ref[0, 0, b]
        j = j_ref[0, 0, b]
        r = r_ref[0, 0, b]

        @pl.when(i != j)
        def _():
            e = (q_ref[pl.ds(i, 1), 0:1] + q_ref[pl.ds(j, 1), 1:2]
                 + xr_ref[pl.ds(r, 1), :])
            s_ref[pl.ds(i, 1), :] += jnp.exp(_leaky(e))

        return carry

    jax.lax.fori_loop(0, eb, body, 0, unroll=8)


def _gat_spmm_body(i_ref, j_ref, r_ref, q_ref, xr_ref, s_ref, x_ref, out_ref):
    estep = pl.program_id(0)
    nsteps = pl.num_programs(0)
    eb = i_ref.shape[2]

    @pl.when(estep == 0)
    def _():
        out_ref[...] = jnp.zeros_like(out_ref)

    def body(b, carry):
        i = i_ref[0, 0, b]
        j = j_ref[0, 0, b]
        r = r_ref[0, 0, b]

        @pl.when(i != j)
        def _():
            e = (q_ref[pl.ds(i, 1), 0:1] + q_ref[pl.ds(j, 1), 1:2]
                 + xr_ref[pl.ds(r, 1), :])
            a = jnp.exp(_leaky(e)) / (s_ref[pl.ds(i, 1), :] + 1e-16)
            out_ref[pl.ds(i, 1), :] += a * x_ref[pl.ds(j, 1), :]

        return carry

    jax.lax.fori_loop(0, eb, body, 0, unroll=4)

    @pl.when(estep == nsteps - 1)
    def _():
        out_ref[...] = jnp.maximum(out_ref[...], 0.0)


def _gat(i3, j3, r3, q, xr, x):
    nb, _, eb = i3.shape
    n, d = x.shape
    vm = pl.BlockSpec(memory_space=pltpu.VMEM)
    s = pl.pallas_call(
        _gat_sum_body,
        grid=(nb,),
        in_specs=[_idx_spec(nb, eb), _idx_spec(nb, eb), _idx_spec(nb, eb),
                  vm, vm],
        out_specs=vm,
        out_shape=jax.ShapeDtypeStruct((n, 1), jnp.float32),
    )(i3, j3, r3, q, xr)
    cb = 512 if d % 512 == 0 else d
    outs = []
    for c0 in range(0, d, cb):
        outs.append(pl.pallas_call(
            _gat_spmm_body,
            grid=(nb,),
            in_specs=[_idx_spec(nb, eb), _idx_spec(nb, eb), _idx_spec(nb, eb),
                      vm, vm, vm, vm],
            out_specs=vm,
            out_shape=jax.ShapeDtypeStruct((n, cb), jnp.float32),
        )(i3, j3, r3, q, xr, s, x[:, c0:c0 + cb]))
    return outs[0] if len(outs) == 1 else jnp.concatenate(outs, axis=1)


def kernel(x_e, edge_index, rel, edge_index_all, rel_all,
           line_graph_index_out, line_graph_val_out,
           line_graph_index_in, line_graph_val_in,
           rel_emb1, rel_emb2, gcn1_w, gcn2_w,
           hw1_w, hw1_b, hw2_w, hw2_b, ww1_w,
           gat_ai, gat_aj, gat_ar, gatr_ai, gatr_aj):
    n, d = x_e.shape
    e_all = edge_index_all.shape[1]
    e_lg = line_graph_index_out.shape[1]
    eb = _edge_blocks(e_all)
    eb_lg = _edge_blocks(e_lg)

    def blk(a, b):
        return a.astype(jnp.int32).reshape(-1, 1, b)

    src3 = blk(edge_index_all[0], eb)
    dst3 = blk(edge_index_all[1], eb)
    rall3 = blk(rel_all, eb)
    rel3 = blk(rel, eb)

    deg, rmax = _deg_relmax(dst3, rel3, n)
    agg1 = _spmm_gcn(dst3, src3, deg, x_e)
    x1 = _gcn_hw(x_e, agg1, gcn1_w, hw1_w, hw1_b)
    agg2 = _spmm_gcn(dst3, src3, deg, x1)
    x2 = _gcn_hw(x1, agg2, gcn2_w, hw2_w, hw2_b)

    re = jnp.where(rmax[0, 0] + 1 == rel_emb1.shape[0], rel_emb1, rel_emb2)
    qr = _proj(re, jnp.stack([gatr_ai, gatr_aj], axis=1))
    lo_src3 = blk(line_graph_index_out[0], eb_lg)
    lo_dst3 = blk(line_graph_index_out[1], eb_lg)
    li_src3 = blk(line_graph_index_in[0], eb_lg)
    li_dst3 = blk(line_graph_index_in[1], eb_lg)
    rel_out = _gat_r(lo_src3, lo_dst3, qr, re)
    rel_in = _gat_r(li_src3, li_dst3, qr, re)
    rel_emb = jnp.concatenate([rel_out, rel_in], axis=0)

    er, pr, xr = _rel_proj(rel_emb, ww1_w[d:d + rel_emb.shape[1]], gat_ar)
    wac = jnp.stack([ww1_w[:d], ww1_w[d + rel_emb.shape[1]:]], axis=1)
    ef, p = _ga_proj(x2, wac)
    outa, outb, outc = _graph_att(src3, dst3, rall3, p, pr, ef, er)
    x_wjq = jnp.concatenate([x2, outa, outb, outc], axis=1)

    q2 = _proj(x_wjq, jnp.stack([gat_ai, gat_aj], axis=1))
    d_wjq = x_wjq.shape[1]
    dpad = -(-d_wjq // 512) * 512 if d_wjq > 512 else d_wjq
    x_in = jnp.pad(x_wjq, ((0, 0), (0, dpad - d_wjq))) if dpad != d_wjq else x_wjq
    gout = _gat(dst3, src3, rall3, q2, xr, x_in)[:, :d_wjq]
    return jnp.concatenate([x_wjq, gout], axis=1)
